# baseline (speedup 1.0000x reference)
import functools

import jax
import jax.numpy as jnp
from jax import lax
from jax.experimental import pallas as pl
from jax.experimental.pallas import tpu as pltpu
from jax.experimental.pallas import tpu_sc as plsc

BATCH = 4096
SEQ = 200
EMBED_DIM = 64
VOCAB = 1000000
NPAIR = VOCAB // 2
NW = 32
BBLK = BATCH // NW
SBLK = 8
N_SBLK = SEQ // SBLK

NBLK_FULL = VOCAB // 128
BLK_PER_W = -(-NBLK_FULL // NW)
TAIL_BASE = (VOCAB - 256)


def _wid():
    return lax.axis_index("s") * 2 + lax.axis_index("c")


def _make_transpose_kernel():
    mesh = plsc.VectorSubcoreMesh(core_axis_name="c", subcore_axis_name="s")

    @functools.partial(
        pl.kernel,
        mesh=mesh,
        out_type=jax.ShapeDtypeStruct((NPAIR, 128), jnp.float32),
        scratch_types=[
            pltpu.VMEM((2, EMBED_DIM, 128), jnp.float32),
            pltpu.VMEM((2, EMBED_DIM, 128), jnp.float32),
            pltpu.SemaphoreType.DMA,
            pltpu.SemaphoreType.DMA,
            pltpu.SemaphoreType.DMA,
            pltpu.SemaphoreType.DMA,
        ],
        compiler_params=pltpu.CompilerParams(use_tc_tiling_on_sc=True,
                                             needs_layout_passes=False),
    )
    def trans(t64, tail_t, tview, src, pairb, rs0, rs1, ws0, ws1):
        rsems = (rs0, rs1)
        wsems = (ws0, ws1)
        w = _wid()
        lo = w * BLK_PER_W
        n = lax.min(jnp.int32(NBLK_FULL), lo + BLK_PER_W) - lo

        iota16 = lax.iota(jnp.int32, 16)
        rows = [iota16 + (k * 16) for k in range(4)]

        def read_start(j, buf):
            pltpu.async_copy(t64.at[:, pl.ds(j * 128, 128)], src.at[buf],
                             rsems[buf])

        def read_wait(j, buf):
            pltpu.make_async_copy(t64.at[:, pl.ds(j * 128, 128)],
                                  src.at[buf], rsems[buf]).wait()

        def extract(buf):
            def body(i, c):
                for u in range(2):
                    for k in range(4):
                        v0 = plsc.load_gather(src.at[buf], [rows[k], c])
                        pairb[buf, 2 * i + u, pl.ds(k * 16, 16)] = v0
                        v1 = plsc.load_gather(src.at[buf], [rows[k], c + 1])
                        pairb[buf, 2 * i + u, pl.ds(64 + k * 16, 16)] = v1
                    c = c + 2
                return c

            lax.fori_loop(0, EMBED_DIM // 2, body,
                          jnp.zeros((16,), jnp.int32))

        def write_start(j, buf):
            pltpu.async_copy(pairb.at[buf], tview.at[pl.ds(j * 64, 64)],
                             wsems[buf])

        def write_wait(j, buf):
            pltpu.make_async_copy(pairb.at[buf],
                                  tview.at[pl.ds(j * 64, 64)],
                                  wsems[buf]).wait()

        @pl.when(n > 0)
        def _():
            read_start(lo, 0)

        def pair_steps(g, carry):
            t0 = g * 2

            def do(t, buf):
                @pl.when(t < n)
                def _():
                    j = lo + t

                    @pl.when(t + 1 < n)
                    def _():
                        read_start(j + 1, 1 - buf)
                    read_wait(j, buf)

                    @pl.when(t >= 2)
                    def _():
                        write_wait(j - 2, buf)
                    extract(buf)
                    write_start(j, buf)

            do(t0, 0)
            do(t0 + 1, 1)
            return carry

        lax.fori_loop(0, (BLK_PER_W + 1) // 2, pair_steps, 0)

        def drain(buf):
            @pl.when((n >= 1) & (lax.rem(n - 1, 2) == buf))
            def _():
                write_wait(lo + n - 1, buf)

            @pl.when((n >= 2) & (lax.rem(n - 2, 2) == buf))
            def _():
                write_wait(lo + n - 2, buf)

        drain(0)
        drain(1)

        @pl.when(w == NW - 1)
        def _():
            pltpu.sync_copy(tail_t.at[:, pl.ds(128, 128)], src.at[0])
            extract(0)
            pltpu.sync_copy(pairb.at[0],
                            tview.at[pl.ds((TAIL_BASE + 128) // 2, 64)])

    return trans


def _make_lookup_kernel():
    mesh = plsc.VectorSubcoreMesh(core_axis_name="c", subcore_axis_name="s")

    @functools.partial(
        pl.kernel,
        mesh=mesh,
        out_type=jax.ShapeDtypeStruct((SEQ, EMBED_DIM, BATCH), jnp.float32),
        scratch_types=[
            pltpu.VMEM((SBLK, BBLK), jnp.int32),
            pltpu.VMEM((2, BBLK), jnp.int32),
            pltpu.VMEM((2, BBLK, 128), jnp.float32),
            pltpu.VMEM((2, EMBED_DIM, BBLK), jnp.float32),
            pltpu.SemaphoreType.DMA,
            pltpu.SemaphoreType.DMA,
            pltpu.SemaphoreType.DMA,
            pltpu.SemaphoreType.DMA,
        ],
        compiler_params=pltpu.CompilerParams(use_tc_tiling_on_sc=True,
                                             needs_layout_passes=False),
    )
    def emb(sent_t, tview, out_hbm, idxblk, pairv, gbuf, obuf,
            gsem0, gsem1, wsem0, wsem1):
        gsems = (gsem0, gsem1)
        wsems = (wsem0, wsem1)
        bbase = _wid() * BBLK

        iota16 = lax.iota(jnp.int32, 16)
        rows = [iota16 + (k * 16) for k in range(BBLK // 16)]

        def prep_pair(j, buf):
            for k in range(BBLK // 16):
                v = idxblk[j, pl.ds(k * 16, 16)]
                pairv[buf, pl.ds(k * 16, 16)] = lax.shift_right_logical(v, 1)

        def gather_start(buf):
            pltpu.async_copy(tview.at[pairv.at[buf]], gbuf.at[buf],
                             gsems[buf])

        def gather_wait(buf):
            pltpu.make_async_copy(tview.at[pairv.at[buf]], gbuf.at[buf],
                                  gsems[buf]).wait()

        def extract(j, buf):
            colbase = []
            for k in range(BBLK // 16):
                v = idxblk[j, pl.ds(k * 16, 16)]
                colbase.append(lax.shift_left(
                    lax.bitwise_and(v, jnp.int32(1)), 6))

            def body(i, cols):
                for u in range(2):
                    for k in range(BBLK // 16):
                        vals = plsc.load_gather(gbuf.at[buf],
                                                [rows[k], cols[k]])
                        obuf[buf, 2 * i + u, pl.ds(k * 16, 16)] = vals
                    cols = tuple(c + 1 for c in cols)
                return cols

            lax.fori_loop(0, EMBED_DIM // 2, body, tuple(colbase))

        def write_start(s, buf):
            pltpu.async_copy(
                obuf.at[buf],
                out_hbm.at[s, :, pl.ds(bbase, BBLK)], wsems[buf])

        def write_wait(s, buf):
            pltpu.make_async_copy(
                obuf.at[buf],
                out_hbm.at[s, :, pl.ds(bbase, BBLK)], wsems[buf]).wait()

        def sblock(blk, carry):
            pltpu.sync_copy(
                sent_t.at[pl.ds(blk * SBLK, SBLK), pl.ds(bbase, BBLK)],
                idxblk)
            s0 = blk * SBLK
            prep_pair(0, 0)
            gather_start(0)
            for j in range(SBLK):
                buf = j % 2
                nbuf = 1 - buf
                if j + 1 < SBLK:
                    prep_pair(j + 1, nbuf)
                    gather_start(nbuf)
                gather_wait(buf)
                if j >= 2:
                    write_wait(s0 + j - 2, buf)
                extract(j, buf)
                write_start(s0 + j, buf)
            write_wait(s0 + SBLK - 2, 0)
            write_wait(s0 + SBLK - 1, 1)
            return carry

        lax.fori_loop(0, N_SBLK, sblock, 0)

    return emb


_transpose = _make_transpose_kernel()
_lookup = _make_lookup_kernel()


def kernel(sentence, table):
    t64 = jnp.transpose(table)
    tail_t = jnp.transpose(table[TAIL_BASE:])
    sent_t = jnp.transpose(sentence)
    tview = _transpose(t64, tail_t)
    y = _lookup(sent_t, tview)
    return jnp.transpose(y, (2, 0, 1))


# speedup vs baseline: 1.8367x; 1.8367x over previous
import functools

import jax
import jax.numpy as jnp
from jax import lax
from jax.experimental import pallas as pl
from jax.experimental.pallas import tpu as pltpu
from jax.experimental.pallas import tpu_sc as plsc

BATCH = 4096
SEQ = 200
EMBED_DIM = 64
VOCAB = 1000000
NPAIR = VOCAB // 2
NW = 32
BBLK = BATCH // NW
SBLK = 8
N_SBLK = SEQ // SBLK

NBLK_FULL = VOCAB // 128
BLK_PER_W = -(-NBLK_FULL // NW)
TAIL_BASE = (VOCAB - 256)


def _wid():
    return lax.axis_index("s") * 2 + lax.axis_index("c")


def _make_transpose_kernel():
    mesh = plsc.VectorSubcoreMesh(core_axis_name="c", subcore_axis_name="s")

    @functools.partial(
        pl.kernel,
        mesh=mesh,
        out_type=jax.ShapeDtypeStruct((NPAIR, 128), jnp.float32),
        scratch_types=[
            pltpu.VMEM((2, EMBED_DIM, 128), jnp.float32),
            pltpu.VMEM((2, EMBED_DIM, 128), jnp.float32),
            pltpu.SemaphoreType.DMA,
            pltpu.SemaphoreType.DMA,
            pltpu.SemaphoreType.DMA,
            pltpu.SemaphoreType.DMA,
        ],
        compiler_params=pltpu.CompilerParams(use_tc_tiling_on_sc=True,
                                             needs_layout_passes=False),
    )
    def trans(t64, tail_t, tview, src, pairb, rs0, rs1, ws0, ws1):
        rsems = (rs0, rs1)
        wsems = (ws0, ws1)
        w = _wid()
        lo = w * BLK_PER_W
        n = lax.min(jnp.int32(NBLK_FULL), lo + BLK_PER_W) - lo

        iota16 = lax.iota(jnp.int32, 16)
        rows = [iota16 + (k * 16) for k in range(4)]

        def read_start(j, buf):
            pltpu.async_copy(t64.at[:, pl.ds(j * 128, 128)], src.at[buf],
                             rsems[buf])

        def read_wait(j, buf):
            pltpu.make_async_copy(t64.at[:, pl.ds(j * 128, 128)],
                                  src.at[buf], rsems[buf]).wait()

        def extract(buf):
            @plsc.parallel_loop(0, EMBED_DIM, unroll=4,
                                carry=jnp.zeros((16,), jnp.int32))
            def _body(p, c):
                for k in range(4):
                    v0 = plsc.load_gather(src.at[buf], [rows[k], c])
                    pairb[buf, p, pl.ds(k * 16, 16)] = v0
                    v1 = plsc.load_gather(src.at[buf], [rows[k], c + 1])
                    pairb[buf, p, pl.ds(64 + k * 16, 16)] = v1
                return c + 2

        def write_start(j, buf):
            pltpu.async_copy(pairb.at[buf], tview.at[pl.ds(j * 64, 64)],
                             wsems[buf])

        def write_wait(j, buf):
            pltpu.make_async_copy(pairb.at[buf],
                                  tview.at[pl.ds(j * 64, 64)],
                                  wsems[buf]).wait()

        @pl.when(n > 0)
        def _():
            read_start(lo, 0)

        def pair_steps(g, carry):
            t0 = g * 2

            def do(t, buf):
                @pl.when(t < n)
                def _():
                    j = lo + t

                    @pl.when(t + 1 < n)
                    def _():
                        read_start(j + 1, 1 - buf)
                    read_wait(j, buf)

                    @pl.when(t >= 2)
                    def _():
                        write_wait(j - 2, buf)
                    extract(buf)
                    write_start(j, buf)

            do(t0, 0)
            do(t0 + 1, 1)
            return carry

        lax.fori_loop(0, (BLK_PER_W + 1) // 2, pair_steps, 0)

        def drain(buf):
            @pl.when((n >= 1) & (lax.rem(n - 1, 2) == buf))
            def _():
                write_wait(lo + n - 1, buf)

            @pl.when((n >= 2) & (lax.rem(n - 2, 2) == buf))
            def _():
                write_wait(lo + n - 2, buf)

        drain(0)
        drain(1)

        @pl.when(w == NW - 1)
        def _():
            pltpu.sync_copy(tail_t.at[:, pl.ds(128, 128)], src.at[0])
            extract(0)
            pltpu.sync_copy(pairb.at[0],
                            tview.at[pl.ds((TAIL_BASE + 128) // 2, 64)])

    return trans


def _make_lookup_kernel():
    mesh = plsc.VectorSubcoreMesh(core_axis_name="c", subcore_axis_name="s")

    @functools.partial(
        pl.kernel,
        mesh=mesh,
        out_type=jax.ShapeDtypeStruct((SEQ, EMBED_DIM, BATCH), jnp.float32),
        scratch_types=[
            pltpu.VMEM((SBLK, BBLK), jnp.int32),
            pltpu.VMEM((2, BBLK), jnp.int32),
            pltpu.VMEM((2, BBLK, 128), jnp.float32),
            pltpu.VMEM((2, EMBED_DIM, BBLK), jnp.float32),
            pltpu.SemaphoreType.DMA,
            pltpu.SemaphoreType.DMA,
            pltpu.SemaphoreType.DMA,
            pltpu.SemaphoreType.DMA,
        ],
        compiler_params=pltpu.CompilerParams(use_tc_tiling_on_sc=True,
                                             needs_layout_passes=False),
    )
    def emb(sent_t, tview, out_hbm, idxblk, pairv, gbuf, obuf,
            gsem0, gsem1, wsem0, wsem1):
        gsems = (gsem0, gsem1)
        wsems = (wsem0, wsem1)
        bbase = _wid() * BBLK

        iota16 = lax.iota(jnp.int32, 16)
        rows = [iota16 + (k * 16) for k in range(BBLK // 16)]

        def prep_pair(j, buf):
            for k in range(BBLK // 16):
                v = idxblk[j, pl.ds(k * 16, 16)]
                pairv[buf, pl.ds(k * 16, 16)] = lax.shift_right_logical(v, 1)

        def gather_start(buf):
            pltpu.async_copy(tview.at[pairv.at[buf]], gbuf.at[buf],
                             gsems[buf])

        def gather_wait(buf):
            pltpu.make_async_copy(tview.at[pairv.at[buf]], gbuf.at[buf],
                                  gsems[buf]).wait()

        def extract(j, buf):
            colbase = []
            for k in range(BBLK // 16):
                v = idxblk[j, pl.ds(k * 16, 16)]
                colbase.append(lax.shift_left(
                    lax.bitwise_and(v, jnp.int32(1)), 6))

            @plsc.parallel_loop(0, EMBED_DIM, unroll=4,
                                carry=tuple(colbase))
            def _body(d, cols):
                for k in range(BBLK // 16):
                    vals = plsc.load_gather(gbuf.at[buf],
                                            [rows[k], cols[k]])
                    obuf[buf, d, pl.ds(k * 16, 16)] = vals
                return tuple(c + 1 for c in cols)

        def write_start(s, buf):
            pltpu.async_copy(
                obuf.at[buf],
                out_hbm.at[s, :, pl.ds(bbase, BBLK)], wsems[buf])

        def write_wait(s, buf):
            pltpu.make_async_copy(
                obuf.at[buf],
                out_hbm.at[s, :, pl.ds(bbase, BBLK)], wsems[buf]).wait()

        def sblock(blk, carry):
            pltpu.sync_copy(
                sent_t.at[pl.ds(blk * SBLK, SBLK), pl.ds(bbase, BBLK)],
                idxblk)
            s0 = blk * SBLK
            prep_pair(0, 0)
            gather_start(0)
            for j in range(SBLK):
                buf = j % 2
                nbuf = 1 - buf
                if j + 1 < SBLK:
                    prep_pair(j + 1, nbuf)
                    gather_start(nbuf)
                gather_wait(buf)
                if j >= 2:
                    write_wait(s0 + j - 2, buf)
                extract(j, buf)
                write_start(s0 + j, buf)
            write_wait(s0 + SBLK - 2, 0)
            write_wait(s0 + SBLK - 1, 1)
            return carry

        lax.fori_loop(0, N_SBLK, sblock, 0)

    return emb


_transpose = _make_transpose_kernel()
_lookup = _make_lookup_kernel()


def kernel(sentence, table):
    t64 = jnp.transpose(table)
    tail_t = jnp.transpose(table[TAIL_BASE:])
    sent_t = jnp.transpose(sentence)
    tview = _transpose(t64, tail_t)
    y = _lookup(sent_t, tview)
    return jnp.transpose(y, (2, 0, 1))
